# Initial kernel scaffold; baseline (speedup 1.0000x reference)
import jax
import jax.numpy as jnp
from jax.experimental import pallas as pl


def kernel(ego_embed, edge_index, edge_type, relation_embed, dropout):
    raise NotImplementedError("write your pallas kernel here")


# speedup vs baseline: 2.3183x; 2.3183x over previous
import functools

import jax
import jax.numpy as jnp
from jax import lax
from jax.experimental import pallas as pl
from jax.experimental.pallas import tpu as pltpu
from jax.experimental.pallas import tpu_sc as plsc

N_NODES = 10000
N_PAD = 10240
D = 128
N_EDGES = 320000
CW = 16
NC, NS = 2, 16
NW = NC * NS
E_PER_W = N_EDGES // NW
BE = 80
NB = E_PER_W // BE
ROWS_PER_TILE = N_PAD // NS
ZR = 64


def _agg_body(ego_hbm, tail_hbm, head_hbm, type_hbm, rel_hbm,
              acc_out, cnt_out,
              acc_sh, cnt_sh, tail_v, head_v, type_v, rows_v, rel_v,
              zb_v, zc_v, ones_v, sem1, sem2):
    cid = lax.axis_index("c")
    sid = lax.axis_index("s")
    wid = sid * NC + cid

    zero16 = jnp.zeros((16,), jnp.float32)
    one16 = jnp.ones((16,), jnp.float32)

    @pl.loop(0, ZR * (D // 16))
    def _(i):
        zb_v[i // (D // 16), pl.ds((i % (D // 16)) * 16, 16)] = zero16

    @pl.loop(0, ZR)
    def _(r):
        zc_v[r, :] = zero16

    @pl.loop(0, BE)
    def _(r):
        ones_v[r, :] = one16

    @pl.loop(0, ROWS_PER_TILE // ZR)
    def _(j):
        pltpu.sync_copy(
            zb_v, acc_sh.at[pl.ds(sid * ROWS_PER_TILE + j * ZR, ZR)])
        pltpu.sync_copy(
            zc_v, cnt_sh.at[pl.ds(sid * ROWS_PER_TILE + j * ZR, ZR)])
    plsc.subcore_barrier()

    base_e = wid * E_PER_W

    @pl.loop(0, NB)
    def _(b):
        off = base_e + b * BE
        pltpu.sync_copy(tail_hbm.at[pl.ds(off, BE)], tail_v)
        pltpu.sync_copy(head_hbm.at[pl.ds(off, BE)], head_v)
        pltpu.sync_copy(type_hbm.at[pl.ds(off, BE)], type_v)
        c1 = pltpu.async_copy(ego_hbm.at[tail_v], rows_v, sem1)
        c2 = pltpu.async_copy(rel_hbm.at[type_v], rel_v, sem2)
        c1.wait()
        c2.wait()

        @plsc.parallel_loop(0, BE * (D // 16), unroll=8)
        def _(i):
            e = i // (D // 16)
            jj = (i % (D // 16)) * 16
            rows_v[e, pl.ds(jj, 16)] = (
                rows_v[e, pl.ds(jj, 16)] * rel_v[e, pl.ds(jj, 16)])

        pltpu.sync_copy(rows_v, acc_sh.at[head_v], add=True)
        pltpu.sync_copy(ones_v, cnt_sh.at[head_v], add=True)

    plsc.subcore_barrier()
    r0 = sid * ROWS_PER_TILE
    pltpu.sync_copy(acc_sh.at[pl.ds(r0, ROWS_PER_TILE)],
                    acc_out.at[cid, pl.ds(r0, ROWS_PER_TILE)])
    pltpu.sync_copy(cnt_sh.at[pl.ds(r0, ROWS_PER_TILE)],
                    cnt_out.at[cid, pl.ds(r0, ROWS_PER_TILE)])


_agg_sc = pl.kernel(
    _agg_body,
    out_type=[
        jax.ShapeDtypeStruct((NC, N_PAD, D), jnp.float32),
        jax.ShapeDtypeStruct((NC, N_PAD, CW), jnp.float32),
    ],
    mesh=plsc.VectorSubcoreMesh(core_axis_name="c", subcore_axis_name="s"),
    compiler_params=pltpu.CompilerParams(use_tc_tiling_on_sc=False),
    scratch_types=[
        pltpu.VMEM_SHARED((N_PAD, D), jnp.float32),
        pltpu.VMEM_SHARED((N_PAD, CW), jnp.float32),
        pltpu.VMEM((BE,), jnp.int32),
        pltpu.VMEM((BE,), jnp.int32),
        pltpu.VMEM((BE,), jnp.int32),
        pltpu.VMEM((BE, D), jnp.float32),
        pltpu.VMEM((BE, D), jnp.float32),
        pltpu.VMEM((ZR, D), jnp.float32),
        pltpu.VMEM((ZR, CW), jnp.float32),
        pltpu.VMEM((BE, CW), jnp.float32),
        pltpu.SemaphoreType.DMA,
        pltpu.SemaphoreType.DMA,
    ],
)


ROW_BLK = 1024


def _norm_body(acc_ref, cnt_ref, res_ref, ego_out_ref, res_out_ref):
    a = acc_ref[0] + acc_ref[1]
    c = cnt_ref[0, :, 0:1] + cnt_ref[1, :, 0:1]
    mean = a / jnp.maximum(c, 1.0)
    n = jnp.sqrt(jnp.sum(mean * mean, axis=1, keepdims=True))
    ego = mean / jnp.maximum(n, 1e-12)
    ego_out_ref[...] = ego
    res_out_ref[...] = res_ref[...] + ego


def _norm_tc(acc, cnt, res):
    grid = (N_PAD // ROW_BLK,)
    return pl.pallas_call(
        _norm_body,
        grid=grid,
        in_specs=[
            pl.BlockSpec((NC, ROW_BLK, D), lambda i: (0, i, 0)),
            pl.BlockSpec((NC, ROW_BLK, CW), lambda i: (0, i, 0)),
            pl.BlockSpec((ROW_BLK, D), lambda i: (i, 0)),
        ],
        out_specs=[
            pl.BlockSpec((ROW_BLK, D), lambda i: (i, 0)),
            pl.BlockSpec((ROW_BLK, D), lambda i: (i, 0)),
        ],
        out_shape=[
            jax.ShapeDtypeStruct((N_PAD, D), jnp.float32),
            jax.ShapeDtypeStruct((N_PAD, D), jnp.float32),
        ],
    )(acc, cnt, res)


@jax.jit
def kernel(ego_embed, edge_index, edge_type, relation_embed, dropout):
    head = edge_index[0].astype(jnp.int32)
    tail = edge_index[1].astype(jnp.int32)
    typ = edge_type.astype(jnp.int32)
    ego = jnp.pad(ego_embed, ((0, N_PAD - N_NODES), (0, 0)))
    res = ego
    for _ in range(2):
        acc, cnt = _agg_sc(ego, tail, head, typ, relation_embed)
        ego, res = _norm_tc(acc, cnt, res)
    return res[:N_NODES]
